# baseline (speedup 1.0000x reference)
import functools

import jax
import jax.numpy as jnp
from jax import lax
from jax.experimental import pallas as pl
from jax.experimental.pallas import tpu as pltpu
from jax.experimental.pallas import tpu_sc as plsc

N = 10000
NP = 10240
E = 320000
D_IN = 128
HALF = 64
NSUB = 16
NCORE = 2
CHUNK = 128
EPT = 20480
NCH = EPT // CHUNK
EPAD = EPT * NSUB
NCHH = NCH // 4
ROWS_PT = NP // NSUB
SWEEP = 128
NSWEEP = ROWS_PT // SWEEP

_mesh = plsc.VectorSubcoreMesh(core_axis_name="c", subcore_axis_name="s")


@functools.partial(
    pl.kernel,
    out_type=jax.ShapeDtypeStruct((NCORE * NP, HALF), jnp.float32),
    mesh=_mesh,
    compiler_params=pltpu.CompilerParams(use_tc_tiling_on_sc=False),
    scratch_types=[
        pltpu.VMEM((NCHH, CHUNK), jnp.int32),
        pltpu.VMEM((NCHH, CHUNK), jnp.int32),
        pltpu.VMEM((NCHH, CHUNK), jnp.float32),
        pltpu.VMEM((6 * CHUNK, HALF // 2), jnp.int32),
        pltpu.VMEM((2 * CHUNK, HALF), jnp.float32),
        pltpu.VMEM((SWEEP, HALF), jnp.float32),
        pltpu.VMEM((SWEEP, HALF // 2), jnp.int32),
        pltpu.VMEM_SHARED((NP, HALF), jnp.float32),
        pltpu.VMEM_SHARED((NP, HALF // 2), jnp.int32),
        pltpu.SemaphoreType.DMA,
        pltpu.SemaphoreType.DMA,
    ],
)
def _sc_double_spmm(x_hbm, src_hbm, dst_hbm, w_hbm, out_hbm,
                    src_v, dst_v, w_v, gbig, sbig, zb, zbf, T, S,
                    gsem, ssem):
    c = lax.axis_index("c")
    s = lax.axis_index("s")
    row_base = s * ROWS_PT
    col_base = c * NP

    def _load_idx(h):
        pltpu.sync_copy(src_hbm.at[s, pl.ds(h * NCHH, NCHH)], src_v)
        pltpu.sync_copy(dst_hbm.at[s, pl.ds(h * NCHH, NCHH)], dst_v)
        pltpu.sync_copy(w_hbm.at[s, pl.ds(h * NCHH, NCHH)], w_v)

    def _zero_zb():
        def zbody(i, _):
            r = i // 4
            q = i % 4
            zb[r, pl.ds(q * 16, 16)] = jnp.zeros((16,), jnp.float32)
            return 0
        lax.fori_loop(0, SWEEP * 4, zbody, 0)

    def _fill_table(tab):
        for k in range(NSWEEP):
            pltpu.sync_copy(zb, tab.at[pl.ds(row_base + k * SWEEP, SWEEP)])

    for k in range(NSWEEP):
        pltpu.sync_copy(
            x_hbm.at[pl.ds(col_base + row_base + k * SWEEP, SWEEP)], zbf)
        pltpu.sync_copy(zbf, S.at[pl.ds(row_base + k * SWEEP, SWEEP)])
    _zero_zb()
    _fill_table(T)
    plsc.subcore_barrier()

    NGB = 6
    NSB = 2

    himask = jnp.full((16,), -65536, jnp.int32)

    def _mul_weights(gbase, sbase, j):
        @plsc.parallel_loop(0, CHUNK // 16, 1, unroll=2)
        def _(g):
            wg = w_v[j, pl.ds(g * 16, 16)]
            for i in range(16):
                e = g * 16 + i
                wvec = jnp.broadcast_to(wg[i], (16,))
                for q in range(HALF // 32):
                    word = gbig[gbase + e, pl.ds(q * 16, 16)]
                    lo = lax.bitcast_convert_type(word << 16, jnp.float32)
                    hi = lax.bitcast_convert_type(word & himask, jnp.float32)
                    sbig[sbase + e, pl.ds(q * 32, 16)] = lo * wvec
                    sbig[sbase + e, pl.ds(q * 32 + 16, 16)] = hi * wvec

    def _gslot(b):
        return gbig.at[pl.ds(b * CHUNK, CHUNK)]

    def _sslot(b):
        return sbig.at[pl.ds(b * CHUNK, CHUNK)]

    def _spmm_half(Tsrc, Tdst):
        for b in range(NGB - 1):
            pltpu.async_copy(Tsrc.at[src_v.at[b]], _gslot(b), gsem)

        def cbody(j, _):
            b = lax.rem(j, NGB)
            sbn = lax.rem(j, NSB)
            pltpu.make_async_copy(Tsrc.at[src_v.at[j]], _gslot(b), gsem).wait()

            @pl.when(j + NGB - 1 < NCHH)
            def _():
                pltpu.async_copy(Tsrc.at[src_v.at[j + NGB - 1]],
                                 _gslot(lax.rem(j + NGB - 1, NGB)), gsem)

            @pl.when(j >= NSB)
            def _():
                pltpu.make_async_copy(
                    _sslot(sbn), Tdst.at[dst_v.at[j - NSB]], ssem).wait()

            _mul_weights(b * CHUNK, sbn * CHUNK, j)
            pltpu.async_copy(_sslot(sbn), Tdst.at[dst_v.at[j]], ssem, add=True)
            return 0
        lax.fori_loop(0, NCHH, cbody, 0)
        for t in range(NCHH - NSB, NCHH):
            pltpu.make_async_copy(
                _sslot(t % NSB), Tdst.at[dst_v.at[t]], ssem).wait()

    def _spmm_pass():
        for h in range(4):
            _load_idx(h)
            _spmm_half(S, T)

    _spmm_pass()
    plsc.subcore_barrier()

    for k in range(NSWEEP):
        pltpu.sync_copy(T.at[pl.ds(row_base + k * SWEEP, SWEEP)], zb)

        def pbody(i, _):
            r = i // (HALF // 32)
            q = i % (HALF // 32)
            ai = lax.bitcast_convert_type(zb[r, pl.ds(q * 32, 16)], jnp.int32)
            bi = lax.bitcast_convert_type(zb[r, pl.ds(q * 32 + 16, 16)], jnp.int32)
            ra = lax.shift_right_logical(
                ai + 0x7FFF + (lax.shift_right_logical(ai, 16) & 1), 16)
            rb = (bi + 0x7FFF + (lax.shift_right_logical(bi, 16) & 1)) & himask
            zbf[r, pl.ds(q * 16, 16)] = rb | ra
            return 0
        lax.fori_loop(0, SWEEP * (HALF // 32), pbody, 0)
        pltpu.sync_copy(zbf, S.at[pl.ds(row_base + k * SWEEP, SWEEP)])
    _zero_zb()
    _fill_table(T)
    plsc.subcore_barrier()

    _spmm_pass()
    plsc.subcore_barrier()

    for k in range(NSWEEP):
        pltpu.sync_copy(T.at[pl.ds(row_base + k * SWEEP, SWEEP)], zb)
        pltpu.sync_copy(
            zb, out_hbm.at[pl.ds(col_base + row_base + k * SWEEP, SWEEP)])


BN_ROWS = 2048
NBLK = NP // BN_ROWS


def _mlp_body(ha_ref, hb_ref, w_ref, b_ref, o_ref):
    acc = jnp.dot(ha_ref[...], w_ref[0, 0],
                  preferred_element_type=jnp.float32)
    acc += jnp.dot(hb_ref[...], w_ref[0, 1],
                   preferred_element_type=jnp.float32)
    o_ref[...] = jnp.maximum(acc + b_ref[0], 0.0)


def _final_body(ha_ref, hb_ref, wa_ref, wb_ref, b_ref, o_ref):
    acc = jnp.dot(ha_ref[...], wa_ref[...], preferred_element_type=jnp.float32)
    acc += jnp.dot(hb_ref[...], wb_ref[...], preferred_element_type=jnp.float32)
    o_ref[...] = acc + b_ref[...]


def _tc_mlp(h2, w1q, b1q):
    return pl.pallas_call(
        _mlp_body,
        grid=(NCORE, NBLK),
        in_specs=[
            pl.BlockSpec((BN_ROWS, HALF), lambda c, i: (i, 0)),
            pl.BlockSpec((BN_ROWS, HALF), lambda c, i: (i + NBLK, 0)),
            pl.BlockSpec((1, 2, HALF, HALF), lambda c, i: (c, 0, 0, 0)),
            pl.BlockSpec((1, 1, HALF), lambda c, i: (c, 0, 0)),
        ],
        out_specs=pl.BlockSpec((BN_ROWS, HALF), lambda c, i: (c * NBLK + i, 0)),
        out_shape=jax.ShapeDtypeStruct((NCORE * NP, HALF), jnp.float32),
    )(h2, h2, w1q, b1q)


def _tc_final(h5, w2a, w2b, b2):
    return pl.pallas_call(
        _final_body,
        grid=(NBLK,),
        in_specs=[
            pl.BlockSpec((BN_ROWS, HALF), lambda i: (i, 0)),
            pl.BlockSpec((BN_ROWS, HALF), lambda i: (i + NBLK, 0)),
            pl.BlockSpec((HALF, HALF), lambda i: (0, 0)),
            pl.BlockSpec((HALF, HALF), lambda i: (0, 0)),
            pl.BlockSpec((1, HALF), lambda i: (0, 0)),
        ],
        out_specs=pl.BlockSpec((BN_ROWS, HALF), lambda i: (i, 0)),
        out_shape=jax.ShapeDtypeStruct((NP, HALF), jnp.float32),
    )(h5, h5, w2a, w2b, b2)


def kernel(x, edge_index, edge_weight, W1, b1, gamma, beta,
           running_mean, running_var, W2, b2):
    scale = gamma * lax.rsqrt(running_var + 1e-5)
    W1f = W1 * scale[None, :]
    b1f = ((b1 - running_mean) * scale + beta)[None, :]

    x_pad = jnp.pad(x, ((0, NP - N), (0, 0)))
    x_flat = jnp.concatenate([x_pad[:, :HALF], x_pad[:, HALF:]], axis=0)

    def _ileave_bf16(a):
        r = a.shape[0]
        bf = (a.reshape(r, HALF // 32, 2, 16).transpose(0, 1, 3, 2)
              .reshape(r, HALF // 2, 2).astype(jnp.bfloat16))
        return lax.bitcast_convert_type(bf, jnp.int32)

    pad = EPAD - E
    src3 = jnp.pad(edge_index[0], (0, pad)).reshape(NSUB, NCH, CHUNK)
    dst3 = jnp.pad(edge_index[1], (0, pad)).reshape(NSUB, NCH, CHUNK)
    w3 = jnp.pad(edge_weight, (0, pad)).reshape(NSUB, NCH, CHUNK)

    w1q = (W1f.reshape(2, HALF, 2, HALF).transpose(2, 0, 1, 3))
    b1q = b1f.reshape(2, HALF)[:, None, :]

    h2 = _sc_double_spmm(_ileave_bf16(x_flat), src3, dst3, w3)
    h3 = _tc_mlp(h2, w1q, b1q)
    h5 = _sc_double_spmm(_ileave_bf16(h3), src3, dst3, w3)
    return _tc_final(h5, W2[:HALF], W2[HALF:], b2[None, :])[:N]


# speedup vs baseline: 1.0211x; 1.0211x over previous
import functools

import jax
import jax.numpy as jnp
from jax import lax
from jax.experimental import pallas as pl
from jax.experimental.pallas import tpu as pltpu
from jax.experimental.pallas import tpu_sc as plsc

N = 10000
NP = 10240
E = 320000
D_IN = 128
HALF = 64
NSUB = 16
NCORE = 2
CHUNK = 128
EPT = 20480
NCH = EPT // CHUNK
EPAD = EPT * NSUB
NCHH = NCH // 4
ROWS_PT = NP // NSUB
SWEEP = 128
NSWEEP = ROWS_PT // SWEEP

_mesh = plsc.VectorSubcoreMesh(core_axis_name="c", subcore_axis_name="s")


@functools.partial(
    pl.kernel,
    out_type=jax.ShapeDtypeStruct((NCORE * NP, HALF), jnp.float32),
    mesh=_mesh,
    compiler_params=pltpu.CompilerParams(use_tc_tiling_on_sc=False),
    scratch_types=[
        pltpu.VMEM((NCHH, CHUNK), jnp.int32),
        pltpu.VMEM((NCHH, CHUNK), jnp.int32),
        pltpu.VMEM((NCHH, CHUNK), jnp.float32),
        pltpu.VMEM((4 * CHUNK, HALF // 2), jnp.int32),
        pltpu.VMEM((3 * CHUNK, HALF), jnp.float32),
        pltpu.VMEM((SWEEP, HALF), jnp.float32),
        pltpu.VMEM((SWEEP, HALF // 2), jnp.int32),
        pltpu.VMEM_SHARED((NP, HALF), jnp.float32),
        pltpu.VMEM_SHARED((NP, HALF // 2), jnp.int32),
        pltpu.SemaphoreType.DMA,
        pltpu.SemaphoreType.DMA,
    ],
)
def _sc_double_spmm(x_hbm, src_hbm, dst_hbm, w_hbm, out_hbm,
                    src_v, dst_v, w_v, gbig, sbig, zb, zbf, T, S,
                    gsem, ssem):
    c = lax.axis_index("c")
    s = lax.axis_index("s")
    row_base = s * ROWS_PT
    col_base = c * NP

    def _load_idx(h):
        pltpu.sync_copy(src_hbm.at[s, pl.ds(h * NCHH, NCHH)], src_v)
        pltpu.sync_copy(dst_hbm.at[s, pl.ds(h * NCHH, NCHH)], dst_v)
        pltpu.sync_copy(w_hbm.at[s, pl.ds(h * NCHH, NCHH)], w_v)

    def _zero_zb():
        def zbody(i, _):
            r = i // 4
            q = i % 4
            zb[r, pl.ds(q * 16, 16)] = jnp.zeros((16,), jnp.float32)
            return 0
        lax.fori_loop(0, SWEEP * 4, zbody, 0)

    def _fill_table(tab):
        for k in range(NSWEEP):
            pltpu.sync_copy(zb, tab.at[pl.ds(row_base + k * SWEEP, SWEEP)])

    for k in range(NSWEEP):
        pltpu.sync_copy(
            x_hbm.at[pl.ds(col_base + row_base + k * SWEEP, SWEEP)], zbf)
        pltpu.sync_copy(zbf, S.at[pl.ds(row_base + k * SWEEP, SWEEP)])
    _zero_zb()
    _fill_table(T)
    plsc.subcore_barrier()

    NGB = 4
    NSB = 3

    himask = jnp.full((16,), -65536, jnp.int32)

    def _mul_weights(gbase, sbase, j):
        @plsc.parallel_loop(0, CHUNK // 16, 1, unroll=4)
        def _(g):
            wg = w_v[j, pl.ds(g * 16, 16)]
            for i in range(16):
                e = g * 16 + i
                wvec = jnp.broadcast_to(wg[i], (16,))
                for q in range(HALF // 32):
                    word = gbig[gbase + e, pl.ds(q * 16, 16)]
                    lo = lax.bitcast_convert_type(word << 16, jnp.float32)
                    hi = lax.bitcast_convert_type(word & himask, jnp.float32)
                    sbig[sbase + e, pl.ds(q * 32, 16)] = lo * wvec
                    sbig[sbase + e, pl.ds(q * 32 + 16, 16)] = hi * wvec

    def _gslot(b):
        return gbig.at[pl.ds(b * CHUNK, CHUNK)]

    def _sslot(b):
        return sbig.at[pl.ds(b * CHUNK, CHUNK)]

    def _spmm_half(Tsrc, Tdst):
        for b in range(NGB - 1):
            pltpu.async_copy(Tsrc.at[src_v.at[b]], _gslot(b), gsem)

        def cbody(j, _):
            b = lax.rem(j, NGB)
            sbn = lax.rem(j, NSB)
            pltpu.make_async_copy(Tsrc.at[src_v.at[j]], _gslot(b), gsem).wait()

            @pl.when(j + NGB - 1 < NCHH)
            def _():
                pltpu.async_copy(Tsrc.at[src_v.at[j + NGB - 1]],
                                 _gslot(lax.rem(j + NGB - 1, NGB)), gsem)

            @pl.when(j >= NSB)
            def _():
                pltpu.make_async_copy(
                    _sslot(sbn), Tdst.at[dst_v.at[j - NSB]], ssem).wait()

            _mul_weights(b * CHUNK, sbn * CHUNK, j)
            pltpu.async_copy(_sslot(sbn), Tdst.at[dst_v.at[j]], ssem, add=True)
            return 0
        lax.fori_loop(0, NCHH, cbody, 0)
        for t in range(NCHH - NSB, NCHH):
            pltpu.make_async_copy(
                _sslot(t % NSB), Tdst.at[dst_v.at[t]], ssem).wait()

    def _spmm_pass():
        for h in range(4):
            _load_idx(h)
            _spmm_half(S, T)

    _spmm_pass()
    plsc.subcore_barrier()

    for k in range(NSWEEP):
        pltpu.sync_copy(T.at[pl.ds(row_base + k * SWEEP, SWEEP)], zb)

        def pbody(i, _):
            r = i // (HALF // 32)
            q = i % (HALF // 32)
            ai = lax.bitcast_convert_type(zb[r, pl.ds(q * 32, 16)], jnp.int32)
            bi = lax.bitcast_convert_type(zb[r, pl.ds(q * 32 + 16, 16)], jnp.int32)
            ra = lax.shift_right_logical(
                ai + 0x7FFF + (lax.shift_right_logical(ai, 16) & 1), 16)
            rb = (bi + 0x7FFF + (lax.shift_right_logical(bi, 16) & 1)) & himask
            zbf[r, pl.ds(q * 16, 16)] = rb | ra
            return 0
        lax.fori_loop(0, SWEEP * (HALF // 32), pbody, 0)
        pltpu.sync_copy(zbf, S.at[pl.ds(row_base + k * SWEEP, SWEEP)])
    _zero_zb()
    _fill_table(T)
    plsc.subcore_barrier()

    _spmm_pass()
    plsc.subcore_barrier()

    for k in range(NSWEEP):
        pltpu.sync_copy(T.at[pl.ds(row_base + k * SWEEP, SWEEP)], zb)
        pltpu.sync_copy(
            zb, out_hbm.at[pl.ds(col_base + row_base + k * SWEEP, SWEEP)])


BN_ROWS = 2048
NBLK = NP // BN_ROWS


def _mlp_body(ha_ref, hb_ref, w_ref, b_ref, o_ref):
    acc = jnp.dot(ha_ref[...], w_ref[0, 0],
                  preferred_element_type=jnp.float32)
    acc += jnp.dot(hb_ref[...], w_ref[0, 1],
                   preferred_element_type=jnp.float32)
    o_ref[...] = jnp.maximum(acc + b_ref[0], 0.0)


def _final_body(ha_ref, hb_ref, wa_ref, wb_ref, b_ref, o_ref):
    acc = jnp.dot(ha_ref[...], wa_ref[...], preferred_element_type=jnp.float32)
    acc += jnp.dot(hb_ref[...], wb_ref[...], preferred_element_type=jnp.float32)
    o_ref[...] = acc + b_ref[...]


def _tc_mlp(h2, w1q, b1q):
    return pl.pallas_call(
        _mlp_body,
        grid=(NCORE, NBLK),
        in_specs=[
            pl.BlockSpec((BN_ROWS, HALF), lambda c, i: (i, 0)),
            pl.BlockSpec((BN_ROWS, HALF), lambda c, i: (i + NBLK, 0)),
            pl.BlockSpec((1, 2, HALF, HALF), lambda c, i: (c, 0, 0, 0)),
            pl.BlockSpec((1, 1, HALF), lambda c, i: (c, 0, 0)),
        ],
        out_specs=pl.BlockSpec((BN_ROWS, HALF), lambda c, i: (c * NBLK + i, 0)),
        out_shape=jax.ShapeDtypeStruct((NCORE * NP, HALF), jnp.float32),
    )(h2, h2, w1q, b1q)


def _tc_final(h5, w2a, w2b, b2):
    return pl.pallas_call(
        _final_body,
        grid=(NBLK,),
        in_specs=[
            pl.BlockSpec((BN_ROWS, HALF), lambda i: (i, 0)),
            pl.BlockSpec((BN_ROWS, HALF), lambda i: (i + NBLK, 0)),
            pl.BlockSpec((HALF, HALF), lambda i: (0, 0)),
            pl.BlockSpec((HALF, HALF), lambda i: (0, 0)),
            pl.BlockSpec((1, HALF), lambda i: (0, 0)),
        ],
        out_specs=pl.BlockSpec((BN_ROWS, HALF), lambda i: (i, 0)),
        out_shape=jax.ShapeDtypeStruct((NP, HALF), jnp.float32),
    )(h5, h5, w2a, w2b, b2)


def kernel(x, edge_index, edge_weight, W1, b1, gamma, beta,
           running_mean, running_var, W2, b2):
    scale = gamma * lax.rsqrt(running_var + 1e-5)
    W1f = W1 * scale[None, :]
    b1f = ((b1 - running_mean) * scale + beta)[None, :]

    x_pad = jnp.pad(x, ((0, NP - N), (0, 0)))
    x_flat = jnp.concatenate([x_pad[:, :HALF], x_pad[:, HALF:]], axis=0)

    def _ileave_bf16(a):
        r = a.shape[0]
        bf = (a.reshape(r, HALF // 32, 2, 16).transpose(0, 1, 3, 2)
              .reshape(r, HALF // 2, 2).astype(jnp.bfloat16))
        return lax.bitcast_convert_type(bf, jnp.int32)

    pad = EPAD - E
    src3 = jnp.pad(edge_index[0], (0, pad)).reshape(NSUB, NCH, CHUNK)
    dst3 = jnp.pad(edge_index[1], (0, pad)).reshape(NSUB, NCH, CHUNK)
    w3 = jnp.pad(edge_weight, (0, pad)).reshape(NSUB, NCH, CHUNK)

    w1q = (W1f.reshape(2, HALF, 2, HALF).transpose(2, 0, 1, 3))
    b1q = b1f.reshape(2, HALF)[:, None, :]

    h2 = _sc_double_spmm(_ileave_bf16(x_flat), src3, dst3, w3)
    h3 = _tc_mlp(h2, w1q, b1q)
    h5 = _sc_double_spmm(_ileave_bf16(h3), src3, dst3, w3)
    return _tc_final(h5, W2[:HALF], W2[HALF:], b2[None, :])[:N]


# speedup vs baseline: 1.0499x; 1.0281x over previous
import functools

import jax
import jax.numpy as jnp
from jax import lax
from jax.experimental import pallas as pl
from jax.experimental.pallas import tpu as pltpu
from jax.experimental.pallas import tpu_sc as plsc

N = 10000
NP = 10240
E = 320000
D_IN = 128
HALF = 64
NSUB = 16
NCORE = 2
CHUNK = 128
EPT = 20480
NCH = EPT // CHUNK
EPAD = EPT * NSUB
NCHH = NCH // 4
ROWS_PT = NP // NSUB
SWEEP = 128
NSWEEP = ROWS_PT // SWEEP

_mesh = plsc.VectorSubcoreMesh(core_axis_name="c", subcore_axis_name="s")


@functools.partial(
    pl.kernel,
    out_type=jax.ShapeDtypeStruct((NCORE * NP, HALF), jnp.float32),
    mesh=_mesh,
    compiler_params=pltpu.CompilerParams(use_tc_tiling_on_sc=False),
    scratch_types=[
        pltpu.VMEM((NCHH, CHUNK), jnp.int32),
        pltpu.VMEM((NCHH, CHUNK), jnp.int32),
        pltpu.VMEM((NCHH, CHUNK), jnp.float32),
        pltpu.VMEM((4 * CHUNK, HALF // 2), jnp.int32),
        pltpu.VMEM((3 * CHUNK, HALF), jnp.float32),
        pltpu.VMEM((SWEEP, HALF), jnp.float32),
        pltpu.VMEM((SWEEP, HALF // 2), jnp.int32),
        pltpu.VMEM_SHARED((NP, HALF), jnp.float32),
        pltpu.VMEM_SHARED((NP, HALF // 2), jnp.int32),
        pltpu.SemaphoreType.DMA,
        pltpu.SemaphoreType.DMA,
    ],
)
def _sc_double_spmm(x_hbm, src_hbm, dst_hbm, w_hbm, out_hbm,
                    src_v, dst_v, w_v, gbig, sbig, zb, zbf, T, S,
                    gsem, ssem):
    c = lax.axis_index("c")
    s = lax.axis_index("s")
    row_base = s * ROWS_PT
    col_base = c * NP

    def _load_idx(h):
        pltpu.sync_copy(src_hbm.at[s, pl.ds(h * NCHH, NCHH)], src_v)
        pltpu.sync_copy(dst_hbm.at[s, pl.ds(h * NCHH, NCHH)], dst_v)
        pltpu.sync_copy(w_hbm.at[s, pl.ds(h * NCHH, NCHH)], w_v)

    def _zero_zb():
        def zbody(i, _):
            r = i // 4
            q = i % 4
            zb[r, pl.ds(q * 16, 16)] = jnp.zeros((16,), jnp.float32)
            return 0
        lax.fori_loop(0, SWEEP * 4, zbody, 0)

    def _fill_table(tab):
        for k in range(NSWEEP):
            pltpu.sync_copy(zb, tab.at[pl.ds(row_base + k * SWEEP, SWEEP)])

    for k in range(NSWEEP):
        pltpu.sync_copy(
            x_hbm.at[pl.ds(col_base + row_base + k * SWEEP, SWEEP)], zbf)
        pltpu.sync_copy(zbf, S.at[pl.ds(row_base + k * SWEEP, SWEEP)])
    _zero_zb()
    _fill_table(T)
    plsc.subcore_barrier()

    NGB = 4
    NSB = 3

    himask = jnp.full((16,), -65536, jnp.int32)

    def _mul_weights(gbase, sbase, j):
        @plsc.parallel_loop(0, CHUNK // 16, 1, unroll=4)
        def _(g):
            wg = w_v[j, pl.ds(g * 16, 16)]
            for i in range(16):
                e = g * 16 + i
                wvec = jnp.broadcast_to(wg[i], (16,))
                for q in range(HALF // 32):
                    word = gbig[gbase + e, pl.ds(q * 16, 16)]
                    lo = lax.bitcast_convert_type(word << 16, jnp.float32)
                    hi = lax.bitcast_convert_type(word & himask, jnp.float32)
                    sbig[sbase + e, pl.ds(q * 32, 16)] = lo * wvec
                    sbig[sbase + e, pl.ds(q * 32 + 16, 16)] = hi * wvec

    def _gslot(b):
        return gbig.at[pl.ds(b * CHUNK, CHUNK)]

    def _sslot(b):
        return sbig.at[pl.ds(b * CHUNK, CHUNK)]

    def _spmm_half(Tsrc, Tdst):
        for b in range(NGB - 1):
            pltpu.async_copy(Tsrc.at[src_v.at[b]], _gslot(b), gsem)

        def cbody(j, _):
            b = lax.rem(j, NGB)
            sbn = lax.rem(j, NSB)
            pltpu.make_async_copy(Tsrc.at[src_v.at[j]], _gslot(b), gsem).wait()

            @pl.when(j + NGB - 1 < NCHH)
            def _():
                pltpu.async_copy(Tsrc.at[src_v.at[j + NGB - 1]],
                                 _gslot(lax.rem(j + NGB - 1, NGB)), gsem)

            @pl.when(j >= NSB)
            def _():
                pltpu.make_async_copy(
                    _sslot(sbn), Tdst.at[dst_v.at[j - NSB]], ssem).wait()

            _mul_weights(b * CHUNK, sbn * CHUNK, j)
            pltpu.async_copy(_sslot(sbn), Tdst.at[dst_v.at[j]], ssem, add=True)
            return 0
        lax.fori_loop(0, NCHH, cbody, 0)
        for t in range(NCHH - NSB, NCHH):
            pltpu.make_async_copy(
                _sslot(t % NSB), Tdst.at[dst_v.at[t]], ssem).wait()

    def _spmm_pass():
        for h in range(4):
            _load_idx(h)
            _spmm_half(S, T)

    _spmm_pass()
    plsc.subcore_barrier()

    for k in range(NSWEEP):
        pltpu.sync_copy(T.at[pl.ds(row_base + k * SWEEP, SWEEP)], zb)

        def pbody(i, _):
            r = i // (HALF // 32)
            q = i % (HALF // 32)
            ai = lax.bitcast_convert_type(zb[r, pl.ds(q * 32, 16)], jnp.int32)
            bi = lax.bitcast_convert_type(zb[r, pl.ds(q * 32 + 16, 16)], jnp.int32)
            ra = lax.shift_right_logical(
                ai + 0x7FFF + (lax.shift_right_logical(ai, 16) & 1), 16)
            rb = (bi + 0x7FFF + (lax.shift_right_logical(bi, 16) & 1)) & himask
            zbf[r, pl.ds(q * 16, 16)] = rb | ra
            return 0
        lax.fori_loop(0, SWEEP * (HALF // 32), pbody, 0)
        pltpu.sync_copy(zbf, S.at[pl.ds(row_base + k * SWEEP, SWEEP)])
    _zero_zb()
    _fill_table(T)
    plsc.subcore_barrier()

    _spmm_pass()
    plsc.subcore_barrier()

    for k in range(NSWEEP):
        pltpu.sync_copy(T.at[pl.ds(row_base + k * SWEEP, SWEEP)], zb)
        pltpu.sync_copy(
            zb, out_hbm.at[pl.ds(col_base + row_base + k * SWEEP, SWEEP)])


BN_ROWS = 2048
NBLK = NP // BN_ROWS


def _bf16_pack_words(lo, hi):
    ai = lax.bitcast_convert_type(lo, jnp.int32)
    bi = lax.bitcast_convert_type(hi, jnp.int32)
    ra = lax.shift_right_logical(
        ai + 0x7FFF + (lax.shift_right_logical(ai, 16) & 1), 16)
    rb = (bi + 0x7FFF + (lax.shift_right_logical(bi, 16) & 1)) & (-65536)
    return rb | ra


def _mlp_body(ha_ref, hb_ref, w_ref, b_ref, o_ref):
    acc = jnp.dot(ha_ref[...], w_ref[0, 0],
                  preferred_element_type=jnp.float32)
    acc += jnp.dot(hb_ref[...], w_ref[0, 1],
                   preferred_element_type=jnp.float32)
    h = jnp.maximum(acc + b_ref[0], 0.0)
    o_ref[...] = jnp.concatenate(
        [_bf16_pack_words(h[:, 0:16], h[:, 16:32]),
         _bf16_pack_words(h[:, 32:48], h[:, 48:64])], axis=1)


def _final_body(ha_ref, hb_ref, wa_ref, wb_ref, b_ref, o_ref):
    acc = jnp.dot(ha_ref[...], wa_ref[...], preferred_element_type=jnp.float32)
    acc += jnp.dot(hb_ref[...], wb_ref[...], preferred_element_type=jnp.float32)
    o_ref[...] = acc + b_ref[...]


def _tc_mlp(h2, w1q, b1q):
    return pl.pallas_call(
        _mlp_body,
        grid=(NCORE, NBLK),
        in_specs=[
            pl.BlockSpec((BN_ROWS, HALF), lambda c, i: (i, 0)),
            pl.BlockSpec((BN_ROWS, HALF), lambda c, i: (i + NBLK, 0)),
            pl.BlockSpec((1, 2, HALF, HALF), lambda c, i: (c, 0, 0, 0)),
            pl.BlockSpec((1, 1, HALF), lambda c, i: (c, 0, 0)),
        ],
        out_specs=pl.BlockSpec((BN_ROWS, HALF // 2),
                               lambda c, i: (c * NBLK + i, 0)),
        out_shape=jax.ShapeDtypeStruct((NCORE * NP, HALF // 2), jnp.int32),
    )(h2, h2, w1q, b1q)


def _tc_final(h5, w2a, w2b, b2):
    return pl.pallas_call(
        _final_body,
        grid=(NBLK,),
        in_specs=[
            pl.BlockSpec((BN_ROWS, HALF), lambda i: (i, 0)),
            pl.BlockSpec((BN_ROWS, HALF), lambda i: (i + NBLK, 0)),
            pl.BlockSpec((HALF, HALF), lambda i: (0, 0)),
            pl.BlockSpec((HALF, HALF), lambda i: (0, 0)),
            pl.BlockSpec((1, HALF), lambda i: (0, 0)),
        ],
        out_specs=pl.BlockSpec((BN_ROWS, HALF), lambda i: (i, 0)),
        out_shape=jax.ShapeDtypeStruct((NP, HALF), jnp.float32),
    )(h5, h5, w2a, w2b, b2)


def kernel(x, edge_index, edge_weight, W1, b1, gamma, beta,
           running_mean, running_var, W2, b2):
    scale = gamma * lax.rsqrt(running_var + 1e-5)
    W1f = W1 * scale[None, :]
    b1f = ((b1 - running_mean) * scale + beta)[None, :]

    x_pad = jnp.pad(x, ((0, NP - N), (0, 0)))
    x_flat = jnp.concatenate([x_pad[:, :HALF], x_pad[:, HALF:]], axis=0)

    def _ileave_bf16(a):
        r = a.shape[0]
        bf = (a.reshape(r, HALF // 32, 2, 16).transpose(0, 1, 3, 2)
              .reshape(r, HALF // 2, 2).astype(jnp.bfloat16))
        return lax.bitcast_convert_type(bf, jnp.int32)

    pad = EPAD - E
    src3 = jnp.pad(edge_index[0], (0, pad)).reshape(NSUB, NCH, CHUNK)
    dst3 = jnp.pad(edge_index[1], (0, pad)).reshape(NSUB, NCH, CHUNK)
    w3 = jnp.pad(edge_weight, (0, pad)).reshape(NSUB, NCH, CHUNK)

    w1q = (W1f.reshape(2, HALF, 2, HALF).transpose(2, 0, 1, 3))
    b1q = b1f.reshape(2, HALF)[:, None, :]

    h2 = _sc_double_spmm(_ileave_bf16(x_flat), src3, dst3, w3)
    h3p = _tc_mlp(h2, w1q, b1q)
    h5 = _sc_double_spmm(h3p, src3, dst3, w3)
    return _tc_final(h5, W2[:HALF], W2[HALF:], b2[None, :])[:N]


# speedup vs baseline: 1.0806x; 1.0292x over previous
import functools

import jax
import jax.numpy as jnp
from jax import lax
from jax.experimental import pallas as pl
from jax.experimental.pallas import tpu as pltpu
from jax.experimental.pallas import tpu_sc as plsc

N = 10000
NP = 10240
E = 320000
D_IN = 128
HALF = 64
NSUB = 16
NCORE = 2
CHUNK = 128
EPT = 20480
NCH = EPT // CHUNK
EPAD = EPT * NSUB
NCHH = NCH // 4
ROWS_PT = NP // NSUB
SWEEP = 128
NSWEEP = ROWS_PT // SWEEP

_mesh = plsc.VectorSubcoreMesh(core_axis_name="c", subcore_axis_name="s")


@functools.partial(
    pl.kernel,
    out_type=jax.ShapeDtypeStruct((NCORE * NP, HALF), jnp.float32),
    mesh=_mesh,
    compiler_params=pltpu.CompilerParams(use_tc_tiling_on_sc=False),
    scratch_types=[
        pltpu.VMEM((NCHH, CHUNK), jnp.int32),
        pltpu.VMEM((NCHH, CHUNK), jnp.int32),
        pltpu.VMEM((NCHH, CHUNK), jnp.float32),
        pltpu.VMEM((4 * CHUNK, HALF // 2), jnp.int32),
        pltpu.VMEM((3 * CHUNK, HALF), jnp.float32),
        pltpu.VMEM((SWEEP, HALF), jnp.float32),
        pltpu.VMEM((SWEEP, HALF // 2), jnp.int32),
        pltpu.VMEM_SHARED((NP, HALF), jnp.float32),
        pltpu.VMEM_SHARED((NP, HALF // 2), jnp.int32),
        pltpu.SemaphoreType.DMA,
        pltpu.SemaphoreType.DMA,
    ],
)
def _sc_double_spmm(x_hbm, src_hbm, dst_hbm, w_hbm, out_hbm,
                    src_v, dst_v, w_v, gbig, sbig, zb, zbf, T, S,
                    gsem, ssem):
    c = lax.axis_index("c")
    s = lax.axis_index("s")
    row_base = s * ROWS_PT
    col_base = c * NP

    def _load_idx(h):
        pltpu.sync_copy(src_hbm.at[s, pl.ds(h * NCHH, NCHH)], src_v)
        pltpu.sync_copy(dst_hbm.at[s, pl.ds(h * NCHH, NCHH)], dst_v)
        pltpu.sync_copy(w_hbm.at[s, pl.ds(h * NCHH, NCHH)], w_v)

    def _zero_zb():
        @plsc.parallel_loop(0, SWEEP * 4, 1, unroll=8)
        def _(i):
            r = i // 4
            q = i % 4
            zb[r, pl.ds(q * 16, 16)] = jnp.zeros((16,), jnp.float32)

    def _fill_table(tab):
        for k in range(NSWEEP):
            pltpu.sync_copy(zb, tab.at[pl.ds(row_base + k * SWEEP, SWEEP)])

    for k in range(NSWEEP):
        pltpu.sync_copy(
            x_hbm.at[pl.ds(col_base + row_base + k * SWEEP, SWEEP)], zbf)
        pltpu.sync_copy(zbf, S.at[pl.ds(row_base + k * SWEEP, SWEEP)])
    _zero_zb()
    _fill_table(T)
    plsc.subcore_barrier()

    NGB = 4
    NSB = 3

    himask = jnp.full((16,), -65536, jnp.int32)

    def _mul_weights(gbase, sbase, j):
        @plsc.parallel_loop(0, CHUNK // 16, 1, unroll=4)
        def _(g):
            wg = w_v[j, pl.ds(g * 16, 16)]
            for i in range(16):
                e = g * 16 + i
                wvec = jnp.broadcast_to(wg[i], (16,))
                for q in range(HALF // 32):
                    word = gbig[gbase + e, pl.ds(q * 16, 16)]
                    lo = lax.bitcast_convert_type(word << 16, jnp.float32)
                    hi = lax.bitcast_convert_type(word & himask, jnp.float32)
                    sbig[sbase + e, pl.ds(q * 32, 16)] = lo * wvec
                    sbig[sbase + e, pl.ds(q * 32 + 16, 16)] = hi * wvec

    def _gslot(b):
        return gbig.at[pl.ds(b * CHUNK, CHUNK)]

    def _sslot(b):
        return sbig.at[pl.ds(b * CHUNK, CHUNK)]

    def _spmm_half(Tsrc, Tdst):
        for b in range(NGB - 1):
            pltpu.async_copy(Tsrc.at[src_v.at[b]], _gslot(b), gsem)

        def cbody(j, _):
            b = lax.rem(j, NGB)
            sbn = lax.rem(j, NSB)
            pltpu.make_async_copy(Tsrc.at[src_v.at[j]], _gslot(b), gsem).wait()

            @pl.when(j + NGB - 1 < NCHH)
            def _():
                pltpu.async_copy(Tsrc.at[src_v.at[j + NGB - 1]],
                                 _gslot(lax.rem(j + NGB - 1, NGB)), gsem)

            @pl.when(j >= NSB)
            def _():
                pltpu.make_async_copy(
                    _sslot(sbn), Tdst.at[dst_v.at[j - NSB]], ssem).wait()

            _mul_weights(b * CHUNK, sbn * CHUNK, j)
            pltpu.async_copy(_sslot(sbn), Tdst.at[dst_v.at[j]], ssem, add=True)
            return 0
        lax.fori_loop(0, NCHH, cbody, 0)
        for t in range(NCHH - NSB, NCHH):
            pltpu.make_async_copy(
                _sslot(t % NSB), Tdst.at[dst_v.at[t]], ssem).wait()

    def _spmm_pass():
        for h in range(4):
            _load_idx(h)
            _spmm_half(S, T)

    _spmm_pass()
    plsc.subcore_barrier()

    for k in range(NSWEEP):
        pltpu.sync_copy(T.at[pl.ds(row_base + k * SWEEP, SWEEP)], zb)

        @plsc.parallel_loop(0, SWEEP * (HALF // 32), 1, unroll=4)
        def _(i):
            r = i // (HALF // 32)
            q = i % (HALF // 32)
            ai = lax.bitcast_convert_type(zb[r, pl.ds(q * 32, 16)], jnp.int32)
            bi = lax.bitcast_convert_type(zb[r, pl.ds(q * 32 + 16, 16)], jnp.int32)
            ra = lax.shift_right_logical(
                ai + 0x7FFF + (lax.shift_right_logical(ai, 16) & 1), 16)
            rb = (bi + 0x7FFF + (lax.shift_right_logical(bi, 16) & 1)) & himask
            zbf[r, pl.ds(q * 16, 16)] = rb | ra
        pltpu.sync_copy(zbf, S.at[pl.ds(row_base + k * SWEEP, SWEEP)])
    _zero_zb()
    _fill_table(T)
    plsc.subcore_barrier()

    _spmm_pass()
    plsc.subcore_barrier()

    for k in range(NSWEEP):
        pltpu.sync_copy(T.at[pl.ds(row_base + k * SWEEP, SWEEP)], zb)
        pltpu.sync_copy(
            zb, out_hbm.at[pl.ds(col_base + row_base + k * SWEEP, SWEEP)])


BN_ROWS = 2048
NBLK = NP // BN_ROWS


def _bf16_pack_words(lo, hi):
    ai = lax.bitcast_convert_type(lo, jnp.int32)
    bi = lax.bitcast_convert_type(hi, jnp.int32)
    ra = lax.shift_right_logical(
        ai + 0x7FFF + (lax.shift_right_logical(ai, 16) & 1), 16)
    rb = (bi + 0x7FFF + (lax.shift_right_logical(bi, 16) & 1)) & (-65536)
    return rb | ra


def _mlp_body(ha_ref, hb_ref, w_ref, b_ref, o_ref):
    acc = jnp.dot(ha_ref[...], w_ref[0, 0],
                  preferred_element_type=jnp.float32)
    acc += jnp.dot(hb_ref[...], w_ref[0, 1],
                   preferred_element_type=jnp.float32)
    h = jnp.maximum(acc + b_ref[0], 0.0)
    o_ref[...] = jnp.concatenate(
        [_bf16_pack_words(h[:, 0:16], h[:, 16:32]),
         _bf16_pack_words(h[:, 32:48], h[:, 48:64])], axis=1)


def _final_body(ha_ref, hb_ref, wa_ref, wb_ref, b_ref, o_ref):
    acc = jnp.dot(ha_ref[...], wa_ref[...], preferred_element_type=jnp.float32)
    acc += jnp.dot(hb_ref[...], wb_ref[...], preferred_element_type=jnp.float32)
    o_ref[...] = acc + b_ref[...]


def _tc_mlp(h2, w1q, b1q):
    return pl.pallas_call(
        _mlp_body,
        grid=(NCORE, NBLK),
        in_specs=[
            pl.BlockSpec((BN_ROWS, HALF), lambda c, i: (i, 0)),
            pl.BlockSpec((BN_ROWS, HALF), lambda c, i: (i + NBLK, 0)),
            pl.BlockSpec((1, 2, HALF, HALF), lambda c, i: (c, 0, 0, 0)),
            pl.BlockSpec((1, 1, HALF), lambda c, i: (c, 0, 0)),
        ],
        out_specs=pl.BlockSpec((BN_ROWS, HALF // 2),
                               lambda c, i: (c * NBLK + i, 0)),
        out_shape=jax.ShapeDtypeStruct((NCORE * NP, HALF // 2), jnp.int32),
    )(h2, h2, w1q, b1q)


def _tc_final(h5, w2a, w2b, b2):
    return pl.pallas_call(
        _final_body,
        grid=(NBLK,),
        in_specs=[
            pl.BlockSpec((BN_ROWS, HALF), lambda i: (i, 0)),
            pl.BlockSpec((BN_ROWS, HALF), lambda i: (i + NBLK, 0)),
            pl.BlockSpec((HALF, HALF), lambda i: (0, 0)),
            pl.BlockSpec((HALF, HALF), lambda i: (0, 0)),
            pl.BlockSpec((1, HALF), lambda i: (0, 0)),
        ],
        out_specs=pl.BlockSpec((BN_ROWS, HALF), lambda i: (i, 0)),
        out_shape=jax.ShapeDtypeStruct((NP, HALF), jnp.float32),
    )(h5, h5, w2a, w2b, b2)


def kernel(x, edge_index, edge_weight, W1, b1, gamma, beta,
           running_mean, running_var, W2, b2):
    scale = gamma * lax.rsqrt(running_var + 1e-5)
    W1f = W1 * scale[None, :]
    b1f = ((b1 - running_mean) * scale + beta)[None, :]

    x_pad = jnp.pad(x, ((0, NP - N), (0, 0)))
    x_flat = jnp.concatenate([x_pad[:, :HALF], x_pad[:, HALF:]], axis=0)

    def _ileave_bf16(a):
        r = a.shape[0]
        bf = (a.reshape(r, HALF // 32, 2, 16).transpose(0, 1, 3, 2)
              .reshape(r, HALF // 2, 2).astype(jnp.bfloat16))
        return lax.bitcast_convert_type(bf, jnp.int32)

    pad = EPAD - E
    src3 = jnp.pad(edge_index[0], (0, pad)).reshape(NSUB, NCH, CHUNK)
    dst3 = jnp.pad(edge_index[1], (0, pad)).reshape(NSUB, NCH, CHUNK)
    w3 = jnp.pad(edge_weight, (0, pad)).reshape(NSUB, NCH, CHUNK)

    w1q = (W1f.reshape(2, HALF, 2, HALF).transpose(2, 0, 1, 3))
    b1q = b1f.reshape(2, HALF)[:, None, :]

    h2 = _sc_double_spmm(_ileave_bf16(x_flat), src3, dst3, w3)
    h3p = _tc_mlp(h2, w1q, b1q)
    h5 = _sc_double_spmm(h3p, src3, dst3, w3)
    return _tc_final(h5, W2[:HALF], W2[HALF:], b2[None, :])[:N]
